# Initial kernel scaffold; baseline (speedup 1.0000x reference)
import jax
import jax.numpy as jnp
from jax.experimental import pallas as pl


def kernel(x, edge_index, batch, W0, b0, As, Wh, bh, Wo, bo):
    raise NotImplementedError("write your pallas kernel here")


# speedup vs baseline: 3.8452x; 3.8452x over previous
import functools

import jax
import jax.numpy as jnp
from jax import lax
from jax.experimental import pallas as pl
from jax.experimental.pallas import tpu as pltpu
from jax.experimental.pallas import tpu_sc as plsc

N = 10000
E = 320000
D_IN = 128
HID = 256
HALF = 128
D_OUT = 16
G = 64
T = 20
L = 6

NC = 2
NS = 16
CH = 128
NCHUNK = 160
KSTAGE = 32
NSTAGE = NCHUNK // KSTAGE
E_PAD = NS * NCHUNK * CH
N_ACC = 10112
STRIPE = N_ACC // NS
RB = 1000

_mesh = plsc.VectorSubcoreMesh(core_axis_name="c", subcore_axis_name="s")


@functools.partial(
    pl.kernel,
    out_type=jax.ShapeDtypeStruct((NC, N_ACC, HALF), jnp.float32),
    mesh=_mesh,
    scratch_types=[
        pltpu.VMEM((KSTAGE, CH), jnp.int32),
        pltpu.VMEM((CH, HALF), jnp.float32),
        pltpu.VMEM_SHARED((N_ACC, HALF), jnp.float32),
    ],
)
def _deg_kernel(dst3, zblk, onesh, out, dstbuf, onesv, acc):
    c = lax.axis_index("c")
    sid = lax.axis_index("s")
    row = sid * STRIPE
    pltpu.sync_copy(zblk.at[pl.ds(row, STRIPE)], acc.at[pl.ds(row, STRIPE)])
    pltpu.sync_copy(onesh, onesv)
    plsc.subcore_barrier()

    def stage(st, carry):
        pltpu.sync_copy(dst3.at[sid, pl.ds(st * KSTAGE, KSTAGE)], dstbuf)

        def body(j, carry2):
            pltpu.sync_copy(onesv, acc.at[dstbuf.at[j]], add=True)
            return carry2

        half = KSTAGE // NC
        return lax.fori_loop(c * half, (c + 1) * half, body, carry)

    lax.fori_loop(0, NSTAGE, stage, 0)
    plsc.subcore_barrier()
    pltpu.sync_copy(acc.at[pl.ds(row, STRIPE)], out.at[c, pl.ds(row, STRIPE)])


@functools.partial(
    pl.kernel,
    out_type=jax.ShapeDtypeStruct((NC, N_ACC, HALF), jnp.float32),
    mesh=_mesh,
    scratch_types=[
        pltpu.VMEM((KSTAGE, CH), jnp.int32),
        pltpu.VMEM((KSTAGE, CH), jnp.int32),
        pltpu.VMEM((CH, HALF), jnp.float32),
        pltpu.VMEM_SHARED((N_ACC, HALF), jnp.float32),
        pltpu.SemaphoreType.DMA,
    ],
)
def _prop_kernel(z2, src2, dst3, zblk, out, srcbuf, dstbuf, rowbuf, acc, sem):
    c = lax.axis_index("c")
    sid = lax.axis_index("s")
    row = sid * STRIPE
    pltpu.sync_copy(zblk.at[pl.ds(row, STRIPE)], acc.at[pl.ds(row, STRIPE)])
    plsc.subcore_barrier()

    def stage(st, carry):
        pltpu.sync_copy(src2.at[c, sid, pl.ds(st * KSTAGE, KSTAGE)], srcbuf)
        pltpu.sync_copy(dst3.at[sid, pl.ds(st * KSTAGE, KSTAGE)], dstbuf)

        def body(j, carry2):
            pltpu.async_copy(z2.at[srcbuf.at[j]], rowbuf, sem).wait()
            pltpu.sync_copy(rowbuf, acc.at[dstbuf.at[j]], add=True)
            return carry2

        return lax.fori_loop(0, KSTAGE, body, carry)

    lax.fori_loop(0, NSTAGE, stage, 0)
    plsc.subcore_barrier()
    pltpu.sync_copy(acc.at[pl.ds(row, STRIPE)], out.at[c, pl.ds(row, STRIPE)])


def _expm_body(a_ref, at_ref, u_ref):
    S = a_ref[0] - at_ref[0]
    ii = lax.broadcasted_iota(jnp.int32, (HID, HID), 0)
    jj = lax.broadcasted_iota(jnp.int32, (HID, HID), 1)
    eye = (ii == jj).astype(jnp.float32)
    term = eye
    acc = eye
    for t in range(1, T + 1):
        term = jnp.dot(term, S, preferred_element_type=jnp.float32) / t
        acc = acc + term
    u_ref[0] = acc


def _expm_call(As, Ast):
    return pl.pallas_call(
        _expm_body,
        grid=(L,),
        in_specs=[
            pl.BlockSpec((1, HID, HID), lambda i: (i, 0, 0)),
            pl.BlockSpec((1, HID, HID), lambda i: (i, 0, 0)),
        ],
        out_specs=pl.BlockSpec((1, HID, HID), lambda i: (i, 0, 0)),
        out_shape=jax.ShapeDtypeStruct((L, HID, HID), jnp.float32),
    )(As, Ast)


def _lift_body(x_ref, w_ref, b_ref, d_ref, z_ref, dinv_ref):
    deg = d_ref[0] + d_ref[1]
    dinv = lax.rsqrt(jnp.clip(deg, 1.0, None))
    y = jnp.dot(x_ref[...], w_ref[...], preferred_element_type=jnp.float32)
    z = dinv * (y + b_ref[...])
    z_ref[0] = z[:, :HALF]
    z_ref[1] = z[:, HALF:]
    dinv_ref[...] = dinv


def _lift_call(x, W0, b0r, deg8):
    return pl.pallas_call(
        _lift_body,
        grid=(N // RB,),
        in_specs=[
            pl.BlockSpec((RB, D_IN), lambda i: (i, 0)),
            pl.BlockSpec((D_IN, HID), lambda i: (0, 0)),
            pl.BlockSpec((1, HID), lambda i: (0, 0)),
            pl.BlockSpec((NC, RB, 1), lambda i: (0, i, 0)),
        ],
        out_specs=[
            pl.BlockSpec((NC, RB, HALF), lambda i: (0, i, 0)),
            pl.BlockSpec((RB, 1), lambda i: (i, 0)),
        ],
        out_shape=[
            jax.ShapeDtypeStruct((NC, N, HALF), jnp.float32),
            jax.ShapeDtypeStruct((N, 1), jnp.float32),
        ],
    )(x, W0, b0r, deg8)


def _layer_body(s_ref, dinv_ref, u_ref, z_ref):
    dinv = dinv_ref[...]
    h = jnp.concatenate([s_ref[0], s_ref[1]], axis=1)
    h = jnp.maximum(dinv * h, 0.0)
    z = dinv * jnp.dot(h, u_ref[...], preferred_element_type=jnp.float32)
    z_ref[0] = z[:, :HALF]
    z_ref[1] = z[:, HALF:]


def _layer_call(s, dinv, U):
    return pl.pallas_call(
        _layer_body,
        grid=(N // RB,),
        in_specs=[
            pl.BlockSpec((NC, RB, HALF), lambda i: (0, i, 0)),
            pl.BlockSpec((RB, 1), lambda i: (i, 0)),
            pl.BlockSpec((HID, HID), lambda i: (0, 0)),
        ],
        out_specs=pl.BlockSpec((NC, RB, HALF), lambda i: (0, i, 0)),
        out_shape=jax.ShapeDtypeStruct((NC, N, HALF), jnp.float32),
    )(s, dinv, U)


def _pool_body(s_ref, dinv_ref, b_ref, ps_ref, cnt_ref):
    i = pl.program_id(0)

    @pl.when(i == 0)
    def _():
        ps_ref[...] = jnp.zeros_like(ps_ref)
        cnt_ref[...] = jnp.zeros_like(cnt_ref)

    h = jnp.concatenate([s_ref[0], s_ref[1]], axis=1)
    h = jnp.maximum(dinv_ref[...] * h, 0.0)
    gids = lax.broadcasted_iota(jnp.int32, (1, G), 1)
    onehot = (b_ref[...] == gids).astype(jnp.float32)
    ps_ref[...] += lax.dot_general(
        onehot, h, (((0,), (0,)), ((), ())), preferred_element_type=jnp.float32)
    ones = jnp.ones((RB, 1), jnp.float32)
    cnt_ref[...] += lax.dot_general(
        onehot, ones, (((0,), (0,)), ((), ())), preferred_element_type=jnp.float32)


def _pool_call(s, dinv, batch2):
    return pl.pallas_call(
        _pool_body,
        grid=(N // RB,),
        in_specs=[
            pl.BlockSpec((NC, RB, HALF), lambda i: (0, i, 0)),
            pl.BlockSpec((RB, 1), lambda i: (i, 0)),
            pl.BlockSpec((RB, 1), lambda i: (i, 0)),
        ],
        out_specs=[
            pl.BlockSpec((G, HID), lambda i: (0, 0)),
            pl.BlockSpec((G, 1), lambda i: (0, 0)),
        ],
        out_shape=[
            jax.ShapeDtypeStruct((G, HID), jnp.float32),
            jax.ShapeDtypeStruct((G, 1), jnp.float32),
        ],
    )(s, dinv, batch2)


def _head_body(ps_ref, cnt_ref, wh_ref, bh_ref, wo_ref, bo_ref, o_ref):
    pooled = ps_ref[...] / jnp.clip(cnt_ref[...], 1.0, None)
    hid = jnp.dot(pooled, wh_ref[...], preferred_element_type=jnp.float32)
    hid = jnp.maximum(hid + bh_ref[...], 0.0)
    o_ref[...] = jnp.dot(hid, wo_ref[...],
                         preferred_element_type=jnp.float32) + bo_ref[...]


def _head_call(ps, cnt, Wh, bhr, Wo, bor):
    return pl.pallas_call(
        _head_body,
        out_shape=jax.ShapeDtypeStruct((G, D_OUT), jnp.float32),
    )(ps, cnt, Wh, bhr, Wo, bor)


def kernel(x, edge_index, batch, W0, b0, As, Wh, bh, Wo, bo):
    src = edge_index[0]
    dst = edge_index[1]
    pad = E_PAD - E
    srcp = jnp.pad(src, (0, pad))
    dstp = jnp.pad(dst, (0, pad), constant_values=N)
    src2 = jnp.stack([srcp, srcp + N]).reshape(NC, NS, NCHUNK, CH)
    dst3 = dstp.reshape(NS, NCHUNK, CH)
    zblk = jnp.zeros((N_ACC, HALF), jnp.float32)
    onesh = jnp.ones((CH, HALF), jnp.float32)

    deg2 = _deg_kernel(dst3, zblk, onesh)
    Us = _expm_call(As, jnp.transpose(As, (0, 2, 1)))
    z, dinv = _lift_call(x, W0, b0.reshape(1, HID), deg2[:, :, :1])
    for i in range(L):
        s = _prop_kernel(z.reshape(NC * N, HALF), src2, dst3, zblk)
        z = _layer_call(s, dinv, Us[i])
    s = _prop_kernel(z.reshape(NC * N, HALF), src2, dst3, zblk)
    ps, cnt = _pool_call(s, dinv, batch.reshape(N, 1).astype(jnp.int32))
    return _head_call(ps, cnt, Wh, bh.reshape(1, HID), Wo, bo.reshape(1, D_OUT))


# speedup vs baseline: 4.3691x; 1.1362x over previous
import functools

import jax
import jax.numpy as jnp
from jax import lax
from jax.experimental import pallas as pl
from jax.experimental.pallas import tpu as pltpu
from jax.experimental.pallas import tpu_sc as plsc

N = 10000
E = 320000
D_IN = 128
HID = 256
HALF = 128
D_OUT = 16
G = 64
T = 20
L = 6

NC = 2
NS = 16
CH = 128
NCHUNK = 160
KSTAGE = 32
NSTAGE = NCHUNK // KSTAGE
E_PAD = NS * NCHUNK * CH
N_ACC = 10112
STRIPE = N_ACC // NS
RB = 1000

_mesh = plsc.VectorSubcoreMesh(core_axis_name="c", subcore_axis_name="s")


@functools.partial(
    pl.kernel,
    out_type=jax.ShapeDtypeStruct((NC, N_ACC, HALF), jnp.float32),
    mesh=_mesh,
    scratch_types=[
        pltpu.VMEM((KSTAGE, CH), jnp.int32),
        pltpu.VMEM((CH, HALF), jnp.float32),
        pltpu.VMEM_SHARED((N_ACC, HALF), jnp.float32),
    ],
)
def _deg_kernel(dst3, zblk, onesh, out, dstbuf, onesv, acc):
    c = lax.axis_index("c")
    sid = lax.axis_index("s")
    row = sid * STRIPE
    pltpu.sync_copy(zblk.at[pl.ds(row, STRIPE)], acc.at[pl.ds(row, STRIPE)])
    pltpu.sync_copy(onesh, onesv)
    plsc.subcore_barrier()

    def stage(st, carry):
        pltpu.sync_copy(dst3.at[sid, pl.ds(st * KSTAGE, KSTAGE)], dstbuf)

        def body(j, carry2):
            pltpu.sync_copy(onesv, acc.at[dstbuf.at[j]], add=True)
            return carry2

        half = KSTAGE // NC
        return lax.fori_loop(c * half, (c + 1) * half, body, carry)

    lax.fori_loop(0, NSTAGE, stage, 0)
    plsc.subcore_barrier()
    pltpu.sync_copy(acc.at[pl.ds(row, STRIPE)], out.at[c, pl.ds(row, STRIPE)])


@functools.partial(
    pl.kernel,
    out_type=jax.ShapeDtypeStruct((NC, N_ACC, HALF), jnp.float32),
    mesh=_mesh,
    scratch_types=[
        pltpu.VMEM((KSTAGE, CH), jnp.int32),
        pltpu.VMEM((KSTAGE, CH), jnp.int32),
        pltpu.VMEM((2, CH, HALF), jnp.float32),
        pltpu.VMEM_SHARED((N_ACC, HALF), jnp.float32),
        pltpu.SemaphoreType.DMA,
    ],
)
def _prop_kernel(z2, src2, dst3, zblk, out, srcbuf, dstbuf, rowbuf, acc, gs):
    c = lax.axis_index("c")
    sid = lax.axis_index("s")
    row = sid * STRIPE
    pltpu.sync_copy(zblk.at[pl.ds(row, STRIPE)], acc.at[pl.ds(row, STRIPE)])
    plsc.subcore_barrier()

    def stage(st, carry):
        pltpu.sync_copy(src2.at[c, sid, pl.ds(st * KSTAGE, KSTAGE)], srcbuf)
        pltpu.sync_copy(dst3.at[sid, pl.ds(st * KSTAGE, KSTAGE)], dstbuf)
        pltpu.async_copy(z2.at[srcbuf.at[0]], rowbuf.at[0], gs)

        def body(jj, carry2):
            b = lax.rem(jj, 2)
            pltpu.make_async_copy(z2.at[srcbuf.at[jj]], rowbuf.at[b], gs).wait()

            @pl.when(jj < KSTAGE - 1)
            def _():
                pltpu.async_copy(z2.at[srcbuf.at[jj + 1]], rowbuf.at[1 - b], gs)

            pltpu.sync_copy(rowbuf.at[b], acc.at[dstbuf.at[jj]], add=True)
            return carry2

        return lax.fori_loop(0, KSTAGE, body, carry)

    lax.fori_loop(0, NSTAGE, stage, 0)
    plsc.subcore_barrier()
    pltpu.sync_copy(acc.at[pl.ds(row, STRIPE)], out.at[c, pl.ds(row, STRIPE)])


def _expm_body(a_ref, at_ref, u_ref):
    S = a_ref[0] - at_ref[0]
    ii = lax.broadcasted_iota(jnp.int32, (HID, HID), 0)
    jj = lax.broadcasted_iota(jnp.int32, (HID, HID), 1)
    eye = (ii == jj).astype(jnp.float32)
    term = eye
    acc = eye
    for t in range(1, T + 1):
        term = jnp.dot(term, S, preferred_element_type=jnp.float32) / t
        acc = acc + term
    u_ref[0] = acc


def _expm_call(As, Ast):
    return pl.pallas_call(
        _expm_body,
        grid=(L,),
        in_specs=[
            pl.BlockSpec((1, HID, HID), lambda i: (i, 0, 0)),
            pl.BlockSpec((1, HID, HID), lambda i: (i, 0, 0)),
        ],
        out_specs=pl.BlockSpec((1, HID, HID), lambda i: (i, 0, 0)),
        out_shape=jax.ShapeDtypeStruct((L, HID, HID), jnp.float32),
    )(As, Ast)


def _lift_body(x_ref, w_ref, b_ref, d_ref, z_ref, dinv_ref):
    deg = d_ref[0] + d_ref[1]
    dinv = lax.rsqrt(jnp.clip(deg, 1.0, None))
    y = jnp.dot(x_ref[...], w_ref[...], preferred_element_type=jnp.float32)
    z = dinv * (y + b_ref[...])
    z_ref[0] = z[:, :HALF]
    z_ref[1] = z[:, HALF:]
    dinv_ref[...] = dinv


def _lift_call(x, W0, b0r, deg8):
    return pl.pallas_call(
        _lift_body,
        grid=(N // RB,),
        in_specs=[
            pl.BlockSpec((RB, D_IN), lambda i: (i, 0)),
            pl.BlockSpec((D_IN, HID), lambda i: (0, 0)),
            pl.BlockSpec((1, HID), lambda i: (0, 0)),
            pl.BlockSpec((NC, RB, 1), lambda i: (0, i, 0)),
        ],
        out_specs=[
            pl.BlockSpec((NC, RB, HALF), lambda i: (0, i, 0)),
            pl.BlockSpec((RB, 1), lambda i: (i, 0)),
        ],
        out_shape=[
            jax.ShapeDtypeStruct((NC, N, HALF), jnp.float32),
            jax.ShapeDtypeStruct((N, 1), jnp.float32),
        ],
    )(x, W0, b0r, deg8)


def _layer_body(s_ref, dinv_ref, u_ref, z_ref):
    dinv = dinv_ref[...]
    h = jnp.concatenate([s_ref[0], s_ref[1]], axis=1)
    h = jnp.maximum(dinv * h, 0.0)
    z = dinv * jnp.dot(h, u_ref[...], preferred_element_type=jnp.float32)
    z_ref[0] = z[:, :HALF]
    z_ref[1] = z[:, HALF:]


def _layer_call(s, dinv, U):
    return pl.pallas_call(
        _layer_body,
        grid=(N // RB,),
        in_specs=[
            pl.BlockSpec((NC, RB, HALF), lambda i: (0, i, 0)),
            pl.BlockSpec((RB, 1), lambda i: (i, 0)),
            pl.BlockSpec((HID, HID), lambda i: (0, 0)),
        ],
        out_specs=pl.BlockSpec((NC, RB, HALF), lambda i: (0, i, 0)),
        out_shape=jax.ShapeDtypeStruct((NC, N, HALF), jnp.float32),
    )(s, dinv, U)


def _pool_body(s_ref, dinv_ref, b_ref, ps_ref, cnt_ref):
    i = pl.program_id(0)

    @pl.when(i == 0)
    def _():
        ps_ref[...] = jnp.zeros_like(ps_ref)
        cnt_ref[...] = jnp.zeros_like(cnt_ref)

    h = jnp.concatenate([s_ref[0], s_ref[1]], axis=1)
    h = jnp.maximum(dinv_ref[...] * h, 0.0)
    gids = lax.broadcasted_iota(jnp.int32, (1, G), 1)
    onehot = (b_ref[...] == gids).astype(jnp.float32)
    ps_ref[...] += lax.dot_general(
        onehot, h, (((0,), (0,)), ((), ())), preferred_element_type=jnp.float32)
    ones = jnp.ones((RB, 1), jnp.float32)
    cnt_ref[...] += lax.dot_general(
        onehot, ones, (((0,), (0,)), ((), ())), preferred_element_type=jnp.float32)


def _pool_call(s, dinv, batch2):
    return pl.pallas_call(
        _pool_body,
        grid=(N // RB,),
        in_specs=[
            pl.BlockSpec((NC, RB, HALF), lambda i: (0, i, 0)),
            pl.BlockSpec((RB, 1), lambda i: (i, 0)),
            pl.BlockSpec((RB, 1), lambda i: (i, 0)),
        ],
        out_specs=[
            pl.BlockSpec((G, HID), lambda i: (0, 0)),
            pl.BlockSpec((G, 1), lambda i: (0, 0)),
        ],
        out_shape=[
            jax.ShapeDtypeStruct((G, HID), jnp.float32),
            jax.ShapeDtypeStruct((G, 1), jnp.float32),
        ],
    )(s, dinv, batch2)


def _head_body(ps_ref, cnt_ref, wh_ref, bh_ref, wo_ref, bo_ref, o_ref):
    pooled = ps_ref[...] / jnp.clip(cnt_ref[...], 1.0, None)
    hid = jnp.dot(pooled, wh_ref[...], preferred_element_type=jnp.float32)
    hid = jnp.maximum(hid + bh_ref[...], 0.0)
    o_ref[...] = jnp.dot(hid, wo_ref[...],
                         preferred_element_type=jnp.float32) + bo_ref[...]


def _head_call(ps, cnt, Wh, bhr, Wo, bor):
    return pl.pallas_call(
        _head_body,
        out_shape=jax.ShapeDtypeStruct((G, D_OUT), jnp.float32),
    )(ps, cnt, Wh, bhr, Wo, bor)


def kernel(x, edge_index, batch, W0, b0, As, Wh, bh, Wo, bo):
    src = edge_index[0]
    dst = edge_index[1]
    pad = E_PAD - E
    srcp = jnp.pad(src, (0, pad))
    dstp = jnp.pad(dst, (0, pad), constant_values=N)
    src2 = jnp.stack([srcp, srcp + N]).reshape(NC, NS, NCHUNK, CH)
    dst3 = dstp.reshape(NS, NCHUNK, CH)
    zblk = jnp.zeros((N_ACC, HALF), jnp.float32)
    onesh = jnp.ones((CH, HALF), jnp.float32)

    deg2 = _deg_kernel(dst3, zblk, onesh)
    Us = _expm_call(As, jnp.transpose(As, (0, 2, 1)))
    z, dinv = _lift_call(x, W0, b0.reshape(1, HID), deg2[:, :, :1])
    for i in range(L):
        s = _prop_kernel(z.reshape(NC * N, HALF), src2, dst3, zblk)
        z = _layer_call(s, dinv, Us[i])
    s = _prop_kernel(z.reshape(NC * N, HALF), src2, dst3, zblk)
    ps, cnt = _pool_call(s, dinv, batch.reshape(N, 1).astype(jnp.int32))
    return _head_call(ps, cnt, Wh, bh.reshape(1, HID), Wo, bo.reshape(1, D_OUT))
